# baseline (speedup 1.0000x reference)
import functools

import jax
import jax.numpy as jnp
from jax import lax
from jax.experimental import pallas as pl
from jax.experimental.pallas import tpu as pltpu
from jax.experimental.pallas import tpu_sc as plsc

_MF_DIM = 32
_HIDDEN = 64


def _make_sc_gather(batch, dim, nrows):
    info = plsc.get_sparse_core_info()
    nc, ns = info.num_cores, info.num_subcores
    nw = nc * ns
    assert batch % (8 * nw) == 0
    b_per_w = batch // nw
    mesh = plsc.VectorSubcoreMesh(core_axis_name="c", subcore_axis_name="s")

    @functools.partial(
        pl.kernel,
        mesh=mesh,
        compiler_params=pltpu.CompilerParams(use_tc_tiling_on_sc=False),
        out_type=(
            jax.ShapeDtypeStruct((dim, batch), jnp.float32),
            jax.ShapeDtypeStruct((dim, batch), jnp.float32),
        ),
        scratch_types=[
            pltpu.VMEM((b_per_w,), jnp.int32),
            pltpu.VMEM((b_per_w,), jnp.int32),
            pltpu.VMEM((dim * b_per_w,), jnp.int32),
            pltpu.VMEM((dim * b_per_w,), jnp.int32),
            pltpu.VMEM((dim * b_per_w,), jnp.float32),
            pltpu.VMEM((dim * b_per_w,), jnp.float32),
            pltpu.SemaphoreType.DMA,
            pltpu.SemaphoreType.DMA,
        ],
    )
    def gather_kernel(user_hbm, item_hbm, utab_hbm, itab_hbm, uout_hbm,
                      iout_hbm, uidx_v, iidx_v, ueidx_v, ieidx_v, ubuf,
                      ibuf, usem, isem):
        wid = lax.axis_index("s") * nc + lax.axis_index("c")
        base = wid * b_per_w
        pltpu.sync_copy(user_hbm.at[pl.ds(base, b_per_w)], uidx_v)
        pltpu.sync_copy(item_hbm.at[pl.ds(base, b_per_w)], iidx_v)

        cps = []
        for c in range(dim):
            @pl.loop(0, b_per_w, step=16)
            def _mk(k, c=c):
                off = c * b_per_w + k
                ueidx_v[pl.ds(off, 16)] = uidx_v[pl.ds(k, 16)] + c * nrows
                ieidx_v[pl.ds(off, 16)] = iidx_v[pl.ds(k, 16)] + c * nrows
            cps.append(pltpu.async_copy(
                utab_hbm.at[ueidx_v.at[pl.ds(c * b_per_w, b_per_w)]],
                ubuf.at[pl.ds(c * b_per_w, b_per_w)], usem))
            cps.append(pltpu.async_copy(
                itab_hbm.at[ieidx_v.at[pl.ds(c * b_per_w, b_per_w)]],
                ibuf.at[pl.ds(c * b_per_w, b_per_w)], isem))
        for cp in cps:
            cp.wait()
        for c in range(dim):
            pltpu.sync_copy(ubuf.at[pl.ds(c * b_per_w, b_per_w)],
                            uout_hbm.at[c, pl.ds(base, b_per_w)])
            pltpu.sync_copy(ibuf.at[pl.ds(c * b_per_w, b_per_w)],
                            iout_hbm.at[c, pl.ds(base, b_per_w)])

    return gather_kernel


def _tc_body(u_ref, i_ref, w_ref, b_ref, o_ref):
    s = u_ref[...] + i_ref[...]
    pre = lax.dot_general(
        w_ref[...], s, (((1,), (0,)), ((), ())),
        preferred_element_type=jnp.float32,
    )
    o_ref[...] = jnp.tanh(pre + b_ref[...])


def kernel(user, item, user_table, item_table, W, b):
    batch = user.shape[0]
    nrows = user_table.shape[0]
    gather = _make_sc_gather(batch, _MF_DIM, nrows)
    uT, iT = gather(user, item, user_table.T.reshape(-1),
                    item_table.T.reshape(-1))

    blk = 2048
    hiddenT = pl.pallas_call(
        _tc_body,
        grid=(batch // blk,),
        in_specs=[
            pl.BlockSpec((_MF_DIM, blk), lambda i: (0, i)),
            pl.BlockSpec((_MF_DIM, blk), lambda i: (0, i)),
            pl.BlockSpec((_HIDDEN, _MF_DIM), lambda i: (0, 0)),
            pl.BlockSpec((_HIDDEN, 1), lambda i: (0, 0)),
        ],
        out_specs=pl.BlockSpec((_HIDDEN, blk), lambda i: (0, i)),
        out_shape=jax.ShapeDtypeStruct((_HIDDEN, batch), jnp.float32),
    )(uT, iT, W, b.reshape(_HIDDEN, 1))

    hidden = hiddenT.T.reshape(1, batch, _HIDDEN)
    return hidden, uT.T, iT.T


# speedup vs baseline: 5.4746x; 5.4746x over previous
import jax
import jax.numpy as jnp
from jax import lax
from jax.experimental import pallas as pl
from jax.experimental.pallas import tpu as pltpu
from jax.experimental.pallas import tpu_sc as plsc
import functools

_MF_DIM = 32
_HIDDEN = 64
_PACK = 128 // _MF_DIM


def _make_sc_gather(batch, dim):
    info = plsc.get_sparse_core_info()
    nc, ns = info.num_cores, info.num_subcores
    nw = nc * ns
    assert batch % (8 * nw) == 0
    b_per_w = batch // nw
    mesh = plsc.VectorSubcoreMesh(core_axis_name="c", subcore_axis_name="s")

    @functools.partial(
        pl.kernel,
        mesh=mesh,
        compiler_params=pltpu.CompilerParams(use_tc_tiling_on_sc=False),
        out_type=(
            jax.ShapeDtypeStruct((batch, dim), jnp.float32),
            jax.ShapeDtypeStruct((batch, dim), jnp.float32),
        ),
        scratch_types=[
            pltpu.VMEM((b_per_w,), jnp.int32),
            pltpu.VMEM((b_per_w,), jnp.int32),
            pltpu.VMEM((b_per_w, dim), jnp.float32),
            pltpu.VMEM((b_per_w, dim), jnp.float32),
            pltpu.SemaphoreType.DMA,
            pltpu.SemaphoreType.DMA,
        ],
    )
    def gather_kernel(user_hbm, item_hbm, utab_hbm, itab_hbm, uout_hbm,
                      iout_hbm, uidx_v, iidx_v, urows, irows, usem, isem):
        wid = lax.axis_index("s") * nc + lax.axis_index("c")
        base = wid * b_per_w
        pltpu.sync_copy(user_hbm.at[pl.ds(base, b_per_w)], uidx_v)
        pltpu.sync_copy(item_hbm.at[pl.ds(base, b_per_w)], iidx_v)
        ucp = pltpu.async_copy(utab_hbm.at[uidx_v], urows, usem)
        icp = pltpu.async_copy(itab_hbm.at[iidx_v], irows, isem)
        ucp.wait()
        icp.wait()
        pltpu.sync_copy(urows, uout_hbm.at[pl.ds(base, b_per_w)])
        pltpu.sync_copy(irows, iout_hbm.at[pl.ds(base, b_per_w)])

    return gather_kernel


def _tc_body(u_ref, i_ref, w4_ref, b4_ref, o_ref):
    s = u_ref[...] + i_ref[...]
    pre = lax.dot_general(
        s, w4_ref[...], (((1,), (0,)), ((), ())),
        preferred_element_type=jnp.float32,
    )
    o_ref[...] = jnp.tanh(pre + b4_ref[...])


def kernel(user, item, user_table, item_table, W, b):
    batch = user.shape[0]
    gather = _make_sc_gather(batch, _MF_DIM)
    u_rows, i_rows = gather(user, item, user_table, item_table)
    nrow = batch * _MF_DIM // 128
    uflat = u_rows.reshape(nrow, 128)
    iflat = i_rows.reshape(nrow, 128)
    w4 = jnp.kron(jnp.eye(_PACK, dtype=jnp.float32), W.T)
    b4 = jnp.tile(b, _PACK).reshape(1, _PACK * _HIDDEN)

    rblk = 512
    hidden4 = pl.pallas_call(
        _tc_body,
        grid=(nrow // rblk,),
        in_specs=[
            pl.BlockSpec((rblk, 128), lambda i: (i, 0)),
            pl.BlockSpec((rblk, 128), lambda i: (i, 0)),
            pl.BlockSpec((128, _PACK * _HIDDEN), lambda i: (0, 0)),
            pl.BlockSpec((1, _PACK * _HIDDEN), lambda i: (0, 0)),
        ],
        out_specs=pl.BlockSpec((rblk, _PACK * _HIDDEN), lambda i: (i, 0)),
        out_shape=jax.ShapeDtypeStruct((nrow, _PACK * _HIDDEN), jnp.float32),
    )(uflat, iflat, w4, b4)

    hidden = hidden4.reshape(1, batch, _HIDDEN)
    return hidden, u_rows, i_rows


# speedup vs baseline: 5.5081x; 1.0061x over previous
import jax
import jax.numpy as jnp
from jax import lax
from jax.experimental import pallas as pl
from jax.experimental.pallas import tpu as pltpu
from jax.experimental.pallas import tpu_sc as plsc
import functools

_MF_DIM = 32
_HIDDEN = 64
_PACK = 128 // _MF_DIM


def _make_sc_gather(batch, dim):
    info = plsc.get_sparse_core_info()
    nc, ns = info.num_cores, info.num_subcores
    nw = nc * ns
    assert batch % (8 * nw) == 0
    b_per_w = batch // nw
    mesh = plsc.VectorSubcoreMesh(core_axis_name="c", subcore_axis_name="s")

    @functools.partial(
        pl.kernel,
        mesh=mesh,
        compiler_params=pltpu.CompilerParams(use_tc_tiling_on_sc=False),
        out_type=jax.ShapeDtypeStruct((batch, dim), jnp.float32),
        scratch_types=[
            pltpu.VMEM((b_per_w,), jnp.int32),
            pltpu.VMEM((b_per_w, dim), jnp.float32),
            pltpu.SemaphoreType.DMA,
        ],
    )
    def gather_kernel(idx_hbm, tab_hbm, out_hbm, idx_v, rows, sem):
        wid = lax.axis_index("s") * nc + lax.axis_index("c")
        base = wid * b_per_w
        pltpu.sync_copy(idx_hbm.at[pl.ds(base, b_per_w)], idx_v)
        pltpu.async_copy(tab_hbm.at[idx_v], rows, sem).wait()
        pltpu.sync_copy(rows, out_hbm.at[pl.ds(base, b_per_w)])

    return gather_kernel


def _tc_body(u_ref, i_ref, w4_ref, b4_ref, o_ref):
    s = u_ref[...] + i_ref[...]
    pre = lax.dot_general(
        s, w4_ref[...], (((1,), (0,)), ((), ())),
        preferred_element_type=jnp.float32,
    )
    o_ref[...] = jnp.tanh(pre + b4_ref[...])


def kernel(user, item, user_table, item_table, W, b):
    batch = user.shape[0]
    gather = _make_sc_gather(batch, _MF_DIM)
    u_rows = gather(user, user_table)
    i_rows = gather(item, item_table)
    nrow = batch * _MF_DIM // 128
    uflat = u_rows.reshape(nrow, 128)
    iflat = i_rows.reshape(nrow, 128)
    w4 = jnp.kron(jnp.eye(_PACK, dtype=jnp.float32), W.T)
    b4 = jnp.tile(b, _PACK).reshape(1, _PACK * _HIDDEN)

    rblk = 512
    hidden4 = pl.pallas_call(
        _tc_body,
        grid=(nrow // rblk,),
        in_specs=[
            pl.BlockSpec((rblk, 128), lambda i: (i, 0)),
            pl.BlockSpec((rblk, 128), lambda i: (i, 0)),
            pl.BlockSpec((128, _PACK * _HIDDEN), lambda i: (0, 0)),
            pl.BlockSpec((1, _PACK * _HIDDEN), lambda i: (0, 0)),
        ],
        out_specs=pl.BlockSpec((rblk, _PACK * _HIDDEN), lambda i: (i, 0)),
        out_shape=jax.ShapeDtypeStruct((nrow, _PACK * _HIDDEN), jnp.float32),
    )(uflat, iflat, w4, b4)

    hidden = hidden4.reshape(1, batch, _HIDDEN)
    return hidden, u_rows, i_rows
